# baseline (speedup 1.0000x reference)
import jax
import jax.numpy as jnp
from jax.experimental import pallas as pl
from jax.experimental.pallas import tpu as pltpu

N = 4096
D = 512
K = 128
PBLK = 512
BLK = 1024
NG = 128
SG = 128

_MASK = 0x7FFFFFFF
_GMIN = -2147483647


def _prep_body(x_ref, w_ref, a_ref, out_ref):
    x = x_ref[...]
    fw = jax.nn.softmax(a_ref[...], axis=-1)
    h = jax.lax.dot_general(x, w_ref[...], (((1,), (1,)), ((), ())),
                            preferred_element_type=jnp.float32,
                            precision=jax.lax.Precision.DEFAULT)
    out_ref[...] = jnp.clip(h * fw, -1.0, 1.0)


def _att_body(xb_ref, x_ref, out_ref, y_ref, ncs_ref):
    xf = x_ref[...]

    @pl.when(pl.program_id(0) == 0)
    def _():
        nsq = jax.lax.dot_general(jnp.ones((1, D), jnp.float32), xf * xf,
                                  (((1,), (1,)), ((), ())),
                                  preferred_element_type=jnp.float32,
                                  precision=jax.lax.Precision.HIGHEST)
        ncs_ref[...] = jnp.sqrt(nsq)

    xb = xb_ref[...]
    s = jax.lax.dot_general(xb, xf, (((1,), (1,)), ((), ())),
                            preferred_element_type=jnp.float32,
                            precision=jax.lax.Precision.DEFAULT)
    nr = jnp.sqrt(jnp.sum(xb * xb, axis=1, keepdims=True))
    att = s / (nr * ncs_ref[...])

    ga = att[:, 0:NG]
    for c in range(1, N // NG):
        ga = jnp.maximum(ga, att[:, c * NG:(c + 1) * NG])
    gb = jax.lax.bitcast_convert_type(ga, jnp.int32)
    grp = jnp.where(gb >= 0, gb, gb ^ _MASK)

    thr = []
    for gi in range(BLK // SG):
        sl = slice(gi * SG, (gi + 1) * SG)
        attg = att[sl, :]
        grpg = grp[sl, :]

        def count_ge(t):
            tb = jnp.where(t >= 0, t, t ^ _MASK)
            tv = jax.lax.bitcast_convert_type(tb, jnp.float32)
            return jnp.sum((attg >= tv).astype(jnp.float32), axis=1,
                           keepdims=True)

        g1 = jnp.max(grpg, axis=1, keepdims=True)
        lo = jnp.min(grpg, axis=1, keepdims=True)
        hi = g1 + 1
        clo = count_ge(lo)
        chi = jnp.zeros_like(clo)

        g = jnp.where(grpg >= g1, _GMIN, grpg)
        g2 = jnp.max(g, axis=1, keepdims=True)
        g = jnp.where(g >= g2, _GMIN, g)
        g3 = jnp.max(g, axis=1, keepdims=True)
        g = jnp.where(g >= g3, _GMIN, g)
        g4 = jnp.max(g, axis=1, keepdims=True)
        probe = jnp.clip(g4, lo + 1, hi - 1)
        open_ = (hi - lo) > 1
        c4 = count_ge(probe)
        ge = c4 >= float(K)
        clo = jnp.where(open_ & ge, c4, clo)
        chi = jnp.where(open_ & ~ge, c4, chi)
        lo = jnp.where(open_ & ge, probe, lo)
        hi = jnp.where(open_ & ~ge, probe, hi)

        def _done(lo, hi, clo):
            return (clo == float(K)) | ((hi - lo) <= 128)

        def cond(c):
            lo, hi, clo, _, _ = c
            return jnp.any(~_done(lo, hi, clo))

        def body(c):
            lo, hi, clo, chi, it = c
            span = (hi - lo).astype(jnp.float32)
            frac = (clo - float(K)) / jnp.maximum(clo - chi, 1.0)
            mid_fp = lo + jnp.clip((span * frac).astype(jnp.int32), 1,
                                   hi - 1 - lo)
            mid_bi = (lo >> 1) + (hi >> 1) + (lo & hi & 1)
            mid = jnp.where(it % 2 == 0, mid_fp, mid_bi)
            cnt = count_ge(mid)
            ge = cnt >= float(K)
            act = ~_done(lo, hi, clo)
            upd_lo = act & ge
            upd_hi = act & ~ge
            return (jnp.where(upd_lo, mid, lo), jnp.where(upd_hi, mid, hi),
                    jnp.where(upd_lo, cnt, clo), jnp.where(upd_hi, cnt, chi),
                    it + 1)

        lo, hi, clo, chi, _ = jax.lax.while_loop(
            cond, body, (lo, hi, clo, chi, jnp.int32(0)))
        tb = jnp.where(lo >= 0, lo, lo ^ _MASK)
        thr.append(jax.lax.bitcast_convert_type(tb, jnp.float32))

    tv = jnp.concatenate(thr, axis=0)
    p = jnp.where(att >= tv, jnp.exp(att), 0.0)
    z = jnp.sum(p, axis=1, keepdims=True)
    y = jax.lax.dot_general(p.astype(jnp.bfloat16), out_ref[...],
                            (((1,), (0,)), ((), ())),
                            preferred_element_type=jnp.float32,
                            precision=jax.lax.Precision.DEFAULT)
    y_ref[...] = y / z


def kernel(x, weight, a_param):
    a2 = a_param.reshape(1, D)
    out = pl.pallas_call(
        _prep_body,
        grid=(N // PBLK,),
        in_specs=[pl.BlockSpec((PBLK, D), lambda i: (i, 0)),
                  pl.BlockSpec((D, D), lambda i: (0, 0)),
                  pl.BlockSpec((1, D), lambda i: (0, 0))],
        out_specs=pl.BlockSpec((PBLK, D), lambda i: (i, 0)),
        out_shape=jax.ShapeDtypeStruct((N, D), jnp.float32),
        compiler_params=pltpu.CompilerParams(
            dimension_semantics=("arbitrary",)),
    )(x, weight, a2)
    y = pl.pallas_call(
        _att_body,
        grid=(N // BLK,),
        in_specs=[pl.BlockSpec((BLK, D), lambda i: (i, 0)),
                  pl.BlockSpec((N, D), lambda i: (0, 0)),
                  pl.BlockSpec((N, D), lambda i: (0, 0))],
        out_specs=pl.BlockSpec((BLK, D), lambda i: (i, 0)),
        out_shape=jax.ShapeDtypeStruct((N, D), jnp.float32),
        scratch_shapes=[pltpu.VMEM((1, N), jnp.float32)],
        compiler_params=pltpu.CompilerParams(
            dimension_semantics=("arbitrary",)),
    )(x, x, out)
    return y


# speedup vs baseline: 1.1272x; 1.1272x over previous
import jax
import jax.numpy as jnp
from jax.experimental import pallas as pl
from jax.experimental.pallas import tpu as pltpu

N = 4096
D = 512
K = 128
PBLK = 512
BLK = 1024
NG = 128

_MASK = 0x7FFFFFFF
_GMIN = -2147483647


def _prep_body(x_ref, w_ref, a_ref, out_ref):
    x = x_ref[...]
    fw = jax.nn.softmax(a_ref[...], axis=-1)
    h = jax.lax.dot_general(x, w_ref[...], (((1,), (1,)), ((), ())),
                            preferred_element_type=jnp.float32,
                            precision=jax.lax.Precision.DEFAULT)
    out_ref[...] = jnp.clip(h * fw, -1.0, 1.0)


def _att_body(xb_ref, x_ref, out_ref, y_ref, ncs_ref):
    xf = x_ref[...]

    @pl.when(pl.program_id(0) == 0)
    def _():
        nsq = jax.lax.dot_general(jnp.ones((1, D), jnp.float32), xf * xf,
                                  (((1,), (1,)), ((), ())),
                                  preferred_element_type=jnp.float32,
                                  precision=jax.lax.Precision.HIGHEST)
        ncs_ref[...] = jnp.sqrt(nsq)

    xb = xb_ref[...]
    s = jax.lax.dot_general(xb, xf, (((1,), (1,)), ((), ())),
                            preferred_element_type=jnp.float32,
                            precision=jax.lax.Precision.DEFAULT)
    nr = jnp.sqrt(jnp.sum(xb * xb, axis=1, keepdims=True))
    att = s / (nr * ncs_ref[...])

    def count_ge(t):
        tb = jnp.where(t >= 0, t, t ^ _MASK)
        tv = jax.lax.bitcast_convert_type(tb, jnp.float32)
        return jnp.sum((att >= tv).astype(jnp.float32), axis=1, keepdims=True)

    ga = att[:, 0:NG]
    for c in range(1, N // NG):
        ga = jnp.maximum(ga, att[:, c * NG:(c + 1) * NG])
    gb = jax.lax.bitcast_convert_type(ga, jnp.int32)
    grp = jnp.where(gb >= 0, gb, gb ^ _MASK)
    g1 = jnp.max(grp, axis=1, keepdims=True)
    lo = jnp.min(grp, axis=1, keepdims=True)
    hi = g1 + 1
    clo = count_ge(lo)
    chi = jnp.zeros_like(clo)

    g = jnp.where(grp >= g1, _GMIN, grp)
    g2 = jnp.max(g, axis=1, keepdims=True)
    g = jnp.where(g >= g2, _GMIN, g)
    g3 = jnp.max(g, axis=1, keepdims=True)
    g = jnp.where(g >= g3, _GMIN, g)
    g4 = jnp.max(g, axis=1, keepdims=True)
    probe = jnp.clip(g4, lo + 1, hi - 1)
    open_ = (hi - lo) > 1
    c4 = count_ge(probe)
    ge = c4 >= float(K)
    clo = jnp.where(open_ & ge, c4, clo)
    chi = jnp.where(open_ & ~ge, c4, chi)
    lo = jnp.where(open_ & ge, probe, lo)
    hi = jnp.where(open_ & ~ge, probe, hi)

    def _done(lo, hi, clo):
        return (clo == float(K)) | ((hi - lo) <= 128)

    def cond(c):
        lo, hi, clo, _, _ = c
        return jnp.any(~_done(lo, hi, clo))

    def body(c):
        lo, hi, clo, chi, it = c
        span = (hi - lo).astype(jnp.float32)
        frac = (clo - float(K)) / jnp.maximum(clo - chi, 1.0)
        mid_fp = lo + jnp.clip((span * frac).astype(jnp.int32), 1, hi - 1 - lo)
        mid_bi = (lo >> 1) + (hi >> 1) + (lo & hi & 1)
        mid = jnp.where(it % 2 == 0, mid_fp, mid_bi)
        cnt = count_ge(mid)
        ge = cnt >= float(K)
        act = ~_done(lo, hi, clo)
        upd_lo = act & ge
        upd_hi = act & ~ge
        return (jnp.where(upd_lo, mid, lo), jnp.where(upd_hi, mid, hi),
                jnp.where(upd_lo, cnt, clo), jnp.where(upd_hi, cnt, chi),
                it + 1)

    lo, hi, clo, chi, _ = jax.lax.while_loop(
        cond, body, (lo, hi, clo, chi, jnp.int32(0)))

    tb = jnp.where(lo >= 0, lo, lo ^ _MASK)
    tv = jax.lax.bitcast_convert_type(tb, jnp.float32)
    p = jnp.where(att >= tv, jnp.exp(att), 0.0)
    z = jnp.sum(p, axis=1, keepdims=True)
    y = jax.lax.dot_general(p.astype(jnp.bfloat16), out_ref[...],
                            (((1,), (0,)), ((), ())),
                            preferred_element_type=jnp.float32,
                            precision=jax.lax.Precision.DEFAULT)
    y_ref[...] = y / z


def kernel(x, weight, a_param):
    a2 = a_param.reshape(1, D)
    out = pl.pallas_call(
        _prep_body,
        grid=(N // PBLK,),
        in_specs=[pl.BlockSpec((PBLK, D), lambda i: (i, 0)),
                  pl.BlockSpec((D, D), lambda i: (0, 0)),
                  pl.BlockSpec((1, D), lambda i: (0, 0))],
        out_specs=pl.BlockSpec((PBLK, D), lambda i: (i, 0)),
        out_shape=jax.ShapeDtypeStruct((N, D), jnp.float32),
        compiler_params=pltpu.CompilerParams(
            dimension_semantics=("arbitrary",)),
    )(x, weight, a2)
    y = pl.pallas_call(
        _att_body,
        grid=(N // BLK,),
        in_specs=[pl.BlockSpec((BLK, D), lambda i: (i, 0)),
                  pl.BlockSpec((N, D), lambda i: (0, 0)),
                  pl.BlockSpec((N, D), lambda i: (0, 0))],
        out_specs=pl.BlockSpec((BLK, D), lambda i: (i, 0)),
        out_shape=jax.ShapeDtypeStruct((N, D), jnp.float32),
        scratch_shapes=[pltpu.VMEM((1, N), jnp.float32)],
        compiler_params=pltpu.CompilerParams(
            dimension_semantics=("arbitrary",)),
    )(x, x, out)
    return y


# speedup vs baseline: 1.1498x; 1.0200x over previous
import jax
import jax.numpy as jnp
from jax.experimental import pallas as pl
from jax.experimental.pallas import tpu as pltpu

N = 4096
D = 512
K = 128
PBLK = 512
BLK = 1024
NG = 128

_MASK = 0x7FFFFFFF
_GMIN = -2147483647


def _prep_body(x_ref, w_ref, a_ref, out_ref, ncs_ref):
    x = x_ref[...]
    fw = jax.nn.softmax(a_ref[...], axis=-1)
    h = jax.lax.dot_general(x, w_ref[...], (((1,), (1,)), ((), ())),
                            preferred_element_type=jnp.float32,
                            precision=jax.lax.Precision.DEFAULT)
    out_ref[...] = jnp.clip(h * fw, -1.0, 1.0)
    nsq = jax.lax.dot_general(jnp.ones((1, D), jnp.float32), x * x,
                              (((1,), (1,)), ((), ())),
                              preferred_element_type=jnp.float32,
                              precision=jax.lax.Precision.HIGHEST)
    ncs_ref[...] = jnp.sqrt(nsq)


def _att_body(xb_ref, x_ref, out_ref, ncs_ref, y_ref):
    xf = x_ref[...]
    xb = xb_ref[...]
    s = jax.lax.dot_general(xb, xf, (((1,), (1,)), ((), ())),
                            preferred_element_type=jnp.float32,
                            precision=jax.lax.Precision.DEFAULT)
    nr = jnp.sqrt(jnp.sum(xb * xb, axis=1, keepdims=True))
    att = s / (nr * ncs_ref[...])

    def count_ge(t):
        tb = jnp.where(t >= 0, t, t ^ _MASK)
        tv = jax.lax.bitcast_convert_type(tb, jnp.float32)
        return jnp.sum((att >= tv).astype(jnp.float32), axis=1, keepdims=True)

    ga = att[:, 0:NG]
    for c in range(1, N // NG):
        ga = jnp.maximum(ga, att[:, c * NG:(c + 1) * NG])
    gb = jax.lax.bitcast_convert_type(ga, jnp.int32)
    grp = jnp.where(gb >= 0, gb, gb ^ _MASK)
    g1 = jnp.max(grp, axis=1, keepdims=True)
    lo = jnp.min(grp, axis=1, keepdims=True)
    hi = g1 + 1
    clo = count_ge(lo)
    chi = jnp.zeros_like(clo)

    g = jnp.where(grp >= g1, _GMIN, grp)
    g2 = jnp.max(g, axis=1, keepdims=True)
    g = jnp.where(g >= g2, _GMIN, g)
    g3 = jnp.max(g, axis=1, keepdims=True)
    g = jnp.where(g >= g3, _GMIN, g)
    g4 = jnp.max(g, axis=1, keepdims=True)
    probe = jnp.clip(g4, lo + 1, hi - 1)
    open_ = (hi - lo) > 1
    c4 = count_ge(probe)
    ge = c4 >= float(K)
    clo = jnp.where(open_ & ge, c4, clo)
    chi = jnp.where(open_ & ~ge, c4, chi)
    lo = jnp.where(open_ & ge, probe, lo)
    hi = jnp.where(open_ & ~ge, probe, hi)

    def _done(lo, hi, clo):
        return (clo == float(K)) | ((hi - lo) <= 128)

    def cond(c):
        lo, hi, clo, _, _ = c
        return jnp.any(~_done(lo, hi, clo))

    def body(c):
        lo, hi, clo, chi, it = c
        span = (hi - lo).astype(jnp.float32)
        frac = (clo - float(K)) / jnp.maximum(clo - chi, 1.0)
        mid_fp = lo + jnp.clip((span * frac).astype(jnp.int32), 1, hi - 1 - lo)
        mid_bi = (lo >> 1) + (hi >> 1) + (lo & hi & 1)
        mid = jnp.where(it % 2 == 0, mid_fp, mid_bi)
        cnt = count_ge(mid)
        ge = cnt >= float(K)
        act = ~_done(lo, hi, clo)
        upd_lo = act & ge
        upd_hi = act & ~ge
        return (jnp.where(upd_lo, mid, lo), jnp.where(upd_hi, mid, hi),
                jnp.where(upd_lo, cnt, clo), jnp.where(upd_hi, cnt, chi),
                it + 1)

    lo, hi, clo, chi, _ = jax.lax.while_loop(
        cond, body, (lo, hi, clo, chi, jnp.int32(0)))

    tb = jnp.where(lo >= 0, lo, lo ^ _MASK)
    tv = jax.lax.bitcast_convert_type(tb, jnp.float32)
    p = jnp.where(att >= tv, jnp.exp(att), 0.0)
    z = jnp.sum(p, axis=1, keepdims=True)
    y = jax.lax.dot_general(p.astype(jnp.bfloat16), out_ref[...],
                            (((1,), (0,)), ((), ())),
                            preferred_element_type=jnp.float32,
                            precision=jax.lax.Precision.DEFAULT)
    y_ref[...] = y / z


def kernel(x, weight, a_param):
    a2 = a_param.reshape(1, D)
    out, ncs = pl.pallas_call(
        _prep_body,
        grid=(N // PBLK,),
        in_specs=[pl.BlockSpec((PBLK, D), lambda i: (i, 0)),
                  pl.BlockSpec((D, D), lambda i: (0, 0)),
                  pl.BlockSpec((1, D), lambda i: (0, 0))],
        out_specs=[pl.BlockSpec((PBLK, D), lambda i: (i, 0)),
                   pl.BlockSpec((1, PBLK), lambda i: (0, i))],
        out_shape=[jax.ShapeDtypeStruct((N, D), jnp.float32),
                   jax.ShapeDtypeStruct((1, N), jnp.float32)],
        compiler_params=pltpu.CompilerParams(
            dimension_semantics=("arbitrary",)),
    )(x, weight, a2)
    y = pl.pallas_call(
        _att_body,
        grid=(N // BLK,),
        in_specs=[pl.BlockSpec((BLK, D), lambda i: (i, 0)),
                  pl.BlockSpec((N, D), lambda i: (0, 0)),
                  pl.BlockSpec((N, D), lambda i: (0, 0)),
                  pl.BlockSpec((1, N), lambda i: (0, 0))],
        out_specs=pl.BlockSpec((BLK, D), lambda i: (i, 0)),
        out_shape=jax.ShapeDtypeStruct((N, D), jnp.float32),
        compiler_params=pltpu.CompilerParams(
            dimension_semantics=("arbitrary",)),
    )(x, x, out, ncs)
    return y


# speedup vs baseline: 1.1505x; 1.0007x over previous
import jax
import jax.numpy as jnp
from jax.experimental import pallas as pl
from jax.experimental.pallas import tpu as pltpu

N = 4096
D = 512
K = 128
PBLK = 512
BLK = 1024
NG = 128

_MASK = 0x7FFFFFFF
_GMIN = -2147483647


def _prep_body(x_ref, w_ref, a_ref, out_ref, ncs_ref):
    x = x_ref[...]
    fw = jax.nn.softmax(a_ref[...], axis=-1)
    h = jax.lax.dot_general(x, w_ref[...], (((1,), (1,)), ((), ())),
                            preferred_element_type=jnp.float32,
                            precision=jax.lax.Precision.DEFAULT)
    out_ref[...] = jnp.clip(h * fw, -1.0, 1.0)
    nsq = jax.lax.dot_general(jnp.ones((1, D), jnp.float32), x * x,
                              (((1,), (1,)), ((), ())),
                              preferred_element_type=jnp.float32,
                              precision=jax.lax.Precision.HIGHEST)
    ncs_ref[...] = jnp.sqrt(nsq)


def _att_body(xb_ref, x_ref, out_ref, ncs_ref, y_ref):
    xf = x_ref[...]
    xb = xb_ref[...]
    s = jax.lax.dot_general(xb, xf, (((1,), (1,)), ((), ())),
                            preferred_element_type=jnp.float32,
                            precision=jax.lax.Precision.DEFAULT)
    nr = jnp.sqrt(jnp.sum(xb * xb, axis=1, keepdims=True))
    att = s / (nr * ncs_ref[...])

    def count_ge(t):
        tb = jnp.where(t >= 0, t, t ^ _MASK)
        tv = jax.lax.bitcast_convert_type(tb, jnp.float32)
        return jnp.sum((att >= tv).astype(jnp.float32), axis=1, keepdims=True)

    ga = att[:, 0:NG]
    for c in range(1, N // NG):
        ga = jnp.maximum(ga, att[:, c * NG:(c + 1) * NG])
    gb = jax.lax.bitcast_convert_type(ga, jnp.int32)
    grp = jnp.where(gb >= 0, gb, gb ^ _MASK)
    g1 = jnp.max(grp, axis=1, keepdims=True)
    lo = jnp.min(grp, axis=1, keepdims=True)
    hi = g1 + 1
    clo = count_ge(lo)
    chi = jnp.zeros_like(clo)

    g = jnp.where(grp >= g1, _GMIN, grp)
    g2 = jnp.max(g, axis=1, keepdims=True)
    g = jnp.where(g >= g2, _GMIN, g)
    g3 = jnp.max(g, axis=1, keepdims=True)
    g = jnp.where(g >= g3, _GMIN, g)
    g4 = jnp.max(g, axis=1, keepdims=True)
    probe = jnp.clip(g4, lo + 1, hi - 1)
    open_ = (hi - lo) > 1
    c4 = count_ge(probe)
    ge = c4 >= float(K)
    clo = jnp.where(open_ & ge, c4, clo)
    chi = jnp.where(open_ & ~ge, c4, chi)
    lo = jnp.where(open_ & ge, probe, lo)
    hi = jnp.where(open_ & ~ge, probe, hi)

    def _done(lo, hi, clo):
        return (clo == float(K)) | ((hi - lo) <= 128)

    def cond(c):
        lo, hi, clo, _, _ = c
        return jnp.any(~_done(lo, hi, clo))

    def body(c):
        lo, hi, clo, chi, it = c
        span = (hi - lo).astype(jnp.float32)
        frac = (clo - float(K)) / jnp.maximum(clo - chi, 1.0)
        mid_fp = lo + jnp.clip((span * frac).astype(jnp.int32), 1, hi - 1 - lo)
        mid_bi = (lo >> 1) + (hi >> 1) + (lo & hi & 1)
        mid = jnp.where(it % 2 == 0, mid_fp, mid_bi)
        cnt = count_ge(mid)
        ge = cnt >= float(K)
        act = ~_done(lo, hi, clo)
        upd_lo = act & ge
        upd_hi = act & ~ge
        return (jnp.where(upd_lo, mid, lo), jnp.where(upd_hi, mid, hi),
                jnp.where(upd_lo, cnt, clo), jnp.where(upd_hi, cnt, chi),
                it + 1)

    lo, hi, clo, chi, _ = jax.lax.while_loop(
        cond, body, (lo, hi, clo, chi, jnp.int32(0)))

    tb = jnp.where(lo >= 0, lo, lo ^ _MASK)
    tv = jax.lax.bitcast_convert_type(tb, jnp.float32)
    p = jnp.where(att >= tv, jnp.exp(att), 0.0)
    z = jnp.sum(p, axis=1, keepdims=True)
    y = jax.lax.dot_general(p.astype(jnp.bfloat16), out_ref[...],
                            (((1,), (0,)), ((), ())),
                            preferred_element_type=jnp.float32,
                            precision=jax.lax.Precision.DEFAULT)
    y_ref[...] = y / z


def kernel(x, weight, a_param):
    a2 = a_param.reshape(1, D)
    out, ncs = pl.pallas_call(
        _prep_body,
        grid=(N // PBLK,),
        in_specs=[pl.BlockSpec((PBLK, D), lambda i: (i, 0)),
                  pl.BlockSpec((D, D), lambda i: (0, 0)),
                  pl.BlockSpec((1, D), lambda i: (0, 0))],
        out_specs=[pl.BlockSpec((PBLK, D), lambda i: (i, 0)),
                   pl.BlockSpec((1, PBLK), lambda i: (0, i))],
        out_shape=[jax.ShapeDtypeStruct((N, D), jnp.float32),
                   jax.ShapeDtypeStruct((1, N), jnp.float32)],
        compiler_params=pltpu.CompilerParams(
            dimension_semantics=("parallel",)),
    )(x, weight, a2)
    y = pl.pallas_call(
        _att_body,
        grid=(N // BLK,),
        in_specs=[pl.BlockSpec((BLK, D), lambda i: (i, 0)),
                  pl.BlockSpec((N, D), lambda i: (0, 0)),
                  pl.BlockSpec((N, D), lambda i: (0, 0)),
                  pl.BlockSpec((1, N), lambda i: (0, 0))],
        out_specs=pl.BlockSpec((BLK, D), lambda i: (i, 0)),
        out_shape=jax.ShapeDtypeStruct((N, D), jnp.float32),
        compiler_params=pltpu.CompilerParams(
            dimension_semantics=("parallel",)),
    )(x, x, out, ncs)
    return y
